# baseline (speedup 1.0000x reference)
import functools

import jax
import jax.numpy as jnp
from jax import lax
from jax.experimental import pallas as pl
from jax.experimental.pallas import tpu as pltpu
from jax.experimental.pallas import tpu_sc as plsc

N = 10000
NP = 10240
D = 128
DA = 136
E = 320000
ET = E + N
CH = 64
G = 8
NBLK = 22
WROWS = G * NBLK
NW = 32
EROWS = NW * WROWS
EP = EROWS * CH
NG = CH // 16
NA = 2 * 10016
RPT = NP // 16

_mesh = plsc.VectorSubcoreMesh(core_axis_name="c", subcore_axis_name="s")


def _mm_first_body(x_ref, w_ref, h_ref, al_ref):
    h = jnp.dot(x_ref[...], w_ref[...], preferred_element_type=jnp.float32)
    h_ref[...] = h[:, :D]
    al_ref[...] = h[:, D:D + 2]


def _finish_layer(nd_ref, den_ref, b_ref):
    t = nd_ref[0] + nd_ref[1]
    ones = jnp.ones((NW, 1), jnp.float32)
    dsum = lax.dot_general(den_ref[...], ones, (((0,), (0,)), ((), ())),
                           preferred_element_type=jnp.float32)
    den = jnp.maximum(dsum, 1e-30)
    return jnp.maximum(t / den + b_ref[...], 0.0)


def _mm_mid_body(nd_ref, den_ref, b_ref, w_ref, h_ref, al_ref):
    x2 = _finish_layer(nd_ref, den_ref, b_ref)
    h = jnp.dot(x2, w_ref[...], preferred_element_type=jnp.float32)
    h_ref[...] = h[:, :D]
    al_ref[...] = h[:, D:D + 2]


def _mm_last_body(nd_ref, den_ref, b_ref, w_ref, bo_ref, o_ref):
    x3 = _finish_layer(nd_ref, den_ref, b_ref)
    o_ref[...] = (
        jnp.dot(x3, w_ref[...], preferred_element_type=jnp.float32)
        + bo_ref[...]
    )


def _mm_first(xp, wp):
    return pl.pallas_call(
        _mm_first_body,
        grid=(NP // 512,),
        in_specs=[
            pl.BlockSpec((512, D), lambda i: (i, 0)),
            pl.BlockSpec((D, DA), lambda i: (0, 0)),
        ],
        out_specs=[
            pl.BlockSpec((512, D), lambda i: (i, 0)),
            pl.BlockSpec((512, 2), lambda i: (i, 0)),
        ],
        out_shape=[
            jax.ShapeDtypeStruct((NP, D), jnp.float32),
            jax.ShapeDtypeStruct((NP, 2), jnp.float32),
        ],
    )(xp, wp)


def _mm_mid(nd, den, b, wp):
    return pl.pallas_call(
        _mm_mid_body,
        grid=(NP // 512,),
        in_specs=[
            pl.BlockSpec((2, 512, D), lambda i: (0, i, 0)),
            pl.BlockSpec((NW, 512), lambda i: (0, i)),
            pl.BlockSpec((1, D), lambda i: (0, 0)),
            pl.BlockSpec((D, DA), lambda i: (0, 0)),
        ],
        out_specs=[
            pl.BlockSpec((512, D), lambda i: (i, 0)),
            pl.BlockSpec((512, 2), lambda i: (i, 0)),
        ],
        out_shape=[
            jax.ShapeDtypeStruct((NP, D), jnp.float32),
            jax.ShapeDtypeStruct((NP, 2), jnp.float32),
        ],
    )(nd, den, b, wp)


def _mm_last(nd, den, b, wo, bo):
    return pl.pallas_call(
        _mm_last_body,
        grid=(NP // 512,),
        in_specs=[
            pl.BlockSpec((2, 512, D), lambda i: (0, i, 0)),
            pl.BlockSpec((NW, 512), lambda i: (0, i)),
            pl.BlockSpec((1, D), lambda i: (0, 0)),
            pl.BlockSpec((D, 16), lambda i: (0, 0)),
            pl.BlockSpec((1, 16), lambda i: (0, 0)),
        ],
        out_specs=pl.BlockSpec((512, 16), lambda i: (i, 0)),
        out_shape=jax.ShapeDtypeStruct((NP, 16), jnp.float32),
    )(nd, den, b, wo, bo)


@functools.partial(
    pl.kernel,
    out_type=[
        jax.ShapeDtypeStruct((2, NP, D), jnp.float32),
        jax.ShapeDtypeStruct((NW, NP), jnp.float32),
    ],
    mesh=_mesh,
    compiler_params=pltpu.CompilerParams(
        needs_layout_passes=False, use_tc_tiling_on_sc=False),
    scratch_types=[
        pltpu.VMEM((NA,), jnp.float32),
        pltpu.VMEM((NP,), jnp.float32),
        pltpu.VMEM((G, CH), jnp.int32),
        pltpu.VMEM((G, CH), jnp.int32),
        pltpu.VMEM((G, CH), jnp.int32),
        pltpu.VMEM((G, CH), jnp.int32),
        pltpu.VMEM((CH, D), jnp.float32),
        pltpu.VMEM((CH, D), jnp.float32),
        pltpu.VMEM((CH,), jnp.float32),
        pltpu.VMEM_SHARED((NP, D), jnp.float32),
        pltpu.SemaphoreType.DMA,
        pltpu.SemaphoreType.DMA,
        pltpu.SemaphoreType.DMA,
        pltpu.SemaphoreType.DMA,
        pltpu.SemaphoreType.DMA,
        pltpu.SemaphoreType.DMA,
    ],
)
def _sc_edge(al_hbm, src_hbm, dst_hbm, h_hbm, z_hbm, num_out, den_out,
             al_v, den_t, srcA, dstA, srcB, dstB, rows_e, rows_o, exb,
             num_sp, gsem_e, gsem_o, ssem_e, ssem_o, isemA, isemB):
    c = lax.axis_index("c")
    s = lax.axis_index("s")
    wid = s * 2 + c
    base = wid * WROWS
    zf16 = jnp.zeros((16,), jnp.float32)

    pltpu.sync_copy(al_hbm, al_v)
    pltpu.sync_copy(z_hbm.at[pl.ds(s * RPT, RPT)],
                    num_sp.at[pl.ds(s * RPT, RPT)])

    def zden_body(i, carry):
        den_t[pl.ds(i * 16, 16)] = zf16
        return carry

    lax.fori_loop(0, NP // 16, zden_body, 0)
    plsc.subcore_barrier()

    def _sub_iter(r, i_src, i_dst, rows_x, gsem_x, ssem_x,
                  rows_y, gsem_y, ssem_y, n_src, n_sem, wait_y, wait_next,
                  use_next, next_ok):
        pltpu.make_async_copy(h_hbm.at[i_src.at[0]], rows_x, gsem_x).wait()

        @pl.when(wait_y)
        def _():
            pltpu.make_async_copy(
                rows_y, num_sp.at[i_dst.at[0]], ssem_y).wait()

        @pl.when(wait_next)
        def _():
            pltpu.make_async_copy(
                src_hbm.at[pl.ds(0, G)], n_src[0], n_sem).wait()
            pltpu.make_async_copy(
                src_hbm.at[pl.ds(0, G)], n_src[1], n_sem).wait()

        @pl.when(next_ok & jnp.logical_not(use_next))
        def _():
            pltpu.async_copy(h_hbm.at[i_src.at[jnp.minimum(r + 1, G - 1)]],
                             rows_y, gsem_y)

        @pl.when(next_ok & use_next)
        def _():
            pltpu.async_copy(h_hbm.at[n_src[0].at[0]], rows_y, gsem_y)

        def ex_body(g, carry2):
            srcv = i_src[r, pl.ds(g * 16, 16)]
            dstv = i_dst[r, pl.ds(g * 16, 16)]
            dstv2 = dstv * 2
            as_s = plsc.load_gather(al_v, [srcv * 2])
            as_d = plsc.load_gather(al_v, [dstv2])
            ad_d = plsc.load_gather(al_v, [dstv2 + 1])
            e = as_s + ad_d
            e = jnp.maximum(e, 0.2 * e)
            m = as_d + ad_d
            m = jnp.maximum(m, 0.2 * m)
            exv = jnp.exp(e - m)
            exb[pl.ds(g * 16, 16)] = exv
            plsc.addupdate_scatter(den_t, [dstv], exv)
            return carry2

        lax.fori_loop(0, NG, ex_body, 0)

        def sc_body(g, carry2):
            exv16 = exb[pl.ds(g * 16, 16)]
            for l in range(16):
                i = g * 16 + l
                exq = jnp.full((16,), exv16[l], jnp.float32)
                for f in range(8):
                    rows_x[i, pl.ds(f * 16, 16)] = (
                        rows_x[i, pl.ds(f * 16, 16)] * exq)
            return carry2

        lax.fori_loop(0, NG, sc_body, 0)
        pltpu.async_copy(rows_x, num_sp.at[i_dst.at[r]], ssem_x, add=True)

    def _block(i_src, i_dst, nxt, n_sem, wait0, refill_pending, next_ok):
        true_ = jnp.bool_(True)

        def duo(rp, carry2):
            r = rp * 2
            last = rp >= G // 2 - 1
            _sub_iter(r, i_src, i_dst, rows_e, gsem_e, ssem_e,
                      rows_o, gsem_o, ssem_o, nxt, n_sem,
                      jnp.logical_or(rp > 0, wait0),
                      jnp.logical_and(last, refill_pending),
                      jnp.bool_(False), true_)
            _sub_iter(r + 1, i_src, i_dst, rows_o, gsem_o, ssem_o,
                      rows_e, gsem_e, ssem_e, nxt, n_sem,
                      true_, jnp.bool_(False), last,
                      jnp.logical_or(jnp.logical_not(last), next_ok))
            return carry2

        lax.fori_loop(0, G // 2, duo, 0)

    QMAX = NBLK // 2 - 1

    def pair_body(q, carry):
        blkA = base + q * 2 * G
        _block(srcA, dstA, (srcB, dstB), isemB, jnp.bool_(False),
               jnp.bool_(True), jnp.bool_(True))

        @pl.when(q < QMAX)
        def _():
            pltpu.make_async_copy(
                rows_o, num_sp.at[dstA.at[0]], ssem_o).wait()
            pltpu.async_copy(src_hbm.at[pl.ds(blkA + 2 * G, G)], srcA, isemA)
            pltpu.async_copy(dst_hbm.at[pl.ds(blkA + 2 * G, G)], dstA, isemA)

        _block(srcB, dstB, (srcA, dstA), isemA, q >= QMAX,
               q < QMAX, q < QMAX)

        @pl.when(q < QMAX)
        def _():
            pltpu.make_async_copy(
                rows_o, num_sp.at[dstB.at[0]], ssem_o).wait()
            pltpu.async_copy(src_hbm.at[pl.ds(blkA + 3 * G, G)], srcB, isemB)
            pltpu.async_copy(dst_hbm.at[pl.ds(blkA + 3 * G, G)], dstB, isemB)

        return carry

    pltpu.sync_copy(src_hbm.at[pl.ds(base, G)], srcA)
    pltpu.sync_copy(dst_hbm.at[pl.ds(base, G)], dstA)
    pltpu.async_copy(src_hbm.at[pl.ds(base + G, G)], srcB, isemB)
    pltpu.async_copy(dst_hbm.at[pl.ds(base + G, G)], dstB, isemB)
    pltpu.async_copy(h_hbm.at[srcA.at[0]], rows_e, gsem_e)

    lax.fori_loop(0, NBLK // 2, pair_body, 0)
    pltpu.make_async_copy(rows_o, num_sp.at[dstB.at[0]], ssem_o).wait()
    plsc.subcore_barrier()
    pltpu.sync_copy(num_sp.at[pl.ds(s * RPT, RPT)],
                    num_out.at[c, pl.ds(s * RPT, RPT)])
    pltpu.sync_copy(den_t.at[pl.ds(0, NP)], den_out.at[wid])


def kernel(x, edge_index, W1, a_src1, a_dst1, b1, W2, a_src2, a_dst2, b2,
           Wf, bf, Ws, bs):
    f32 = jnp.float32
    xp = jnp.zeros((NP, D), f32).at[:N].set(x)

    def augment(W, a_src, a_dst):
        return jnp.concatenate(
            [W, (W @ a_src)[:, None], (W @ a_dst)[:, None],
             jnp.zeros((D, DA - D - 2), f32)], axis=1)

    w1p = augment(W1, a_src1, a_dst1)
    w2p = augment(W2, a_src2, a_dst2)
    wo = jnp.concatenate([Wf, Ws, jnp.zeros((D, 6), f32)], axis=1)
    bo = jnp.concatenate([bf, bs, jnp.zeros((6,), f32)])[None, :]

    sl = jnp.arange(N, dtype=jnp.int32)
    pad = jnp.full((EP - ET,), N, jnp.int32)
    src2d = jnp.concatenate([edge_index[0], sl, pad]).reshape(EROWS, CH)
    dst2d = jnp.concatenate([edge_index[1], sl, pad]).reshape(EROWS, CH)
    znd = jnp.zeros((NP, D), f32)

    h1, al1 = _mm_first(xp, w1p)
    nd1, den1 = _sc_edge(al1.reshape(2 * NP)[:NA], src2d, dst2d, h1, znd)
    h2, al2 = _mm_mid(nd1, den1, b1[None, :], w2p)
    nd2, den2 = _sc_edge(al2.reshape(2 * NP)[:NA], src2d, dst2d, h2, znd)
    out = _mm_last(nd2, den2, b2[None, :], wo, bo)
    return (out[:N, :3], out[:N, 3:10])


# speedup vs baseline: 1.0001x; 1.0001x over previous
import functools

import jax
import jax.numpy as jnp
from jax import lax
from jax.experimental import pallas as pl
from jax.experimental.pallas import tpu as pltpu
from jax.experimental.pallas import tpu_sc as plsc

N = 10000
NP = 10240
D = 128
DA = 136
E = 320000
ET = E + N
CH = 64
G = 8
NBLK = 22
WROWS = G * NBLK
NW = 32
EROWS = NW * WROWS
EP = EROWS * CH
NG = CH // 16
NA = 2 * 10016
RPT = NP // 16

_mesh = plsc.VectorSubcoreMesh(core_axis_name="c", subcore_axis_name="s")


def _mm_first_body(x_ref, w_ref, h_ref, al_ref):
    h = jnp.dot(x_ref[...], w_ref[...], preferred_element_type=jnp.float32)
    h_ref[...] = h[:, :D]
    al_ref[...] = h[:, D:D + 2]


def _finish_layer(nd_ref, den_ref, b_ref):
    t = nd_ref[0] + nd_ref[1]
    ones = jnp.ones((NW, 1), jnp.float32)
    dsum = lax.dot_general(den_ref[...], ones, (((0,), (0,)), ((), ())),
                           preferred_element_type=jnp.float32)
    den = jnp.maximum(dsum, 1e-30)
    return jnp.maximum(t / den + b_ref[...], 0.0)


def _mm_mid_body(nd_ref, den_ref, b_ref, w_ref, h_ref, al_ref):
    x2 = _finish_layer(nd_ref, den_ref, b_ref)
    h = jnp.dot(x2, w_ref[...], preferred_element_type=jnp.float32)
    h_ref[...] = h[:, :D]
    al_ref[...] = h[:, D:D + 2]


def _mm_last_body(nd_ref, den_ref, b_ref, w_ref, bo_ref, o_ref):
    x3 = _finish_layer(nd_ref, den_ref, b_ref)
    o_ref[...] = (
        jnp.dot(x3, w_ref[...], preferred_element_type=jnp.float32)
        + bo_ref[...]
    )


def _mm_first(xp, wp):
    return pl.pallas_call(
        _mm_first_body,
        grid=(NP // 512,),
        in_specs=[
            pl.BlockSpec((512, D), lambda i: (i, 0)),
            pl.BlockSpec((D, DA), lambda i: (0, 0)),
        ],
        out_specs=[
            pl.BlockSpec((512, D), lambda i: (i, 0)),
            pl.BlockSpec((512, 2), lambda i: (i, 0)),
        ],
        out_shape=[
            jax.ShapeDtypeStruct((NP, D), jnp.float32),
            jax.ShapeDtypeStruct((NP, 2), jnp.float32),
        ],
    )(xp, wp)


def _mm_mid(nd, den, b, wp):
    return pl.pallas_call(
        _mm_mid_body,
        grid=(NP // 512,),
        in_specs=[
            pl.BlockSpec((2, 512, D), lambda i: (0, i, 0)),
            pl.BlockSpec((NW, 512), lambda i: (0, i)),
            pl.BlockSpec((1, D), lambda i: (0, 0)),
            pl.BlockSpec((D, DA), lambda i: (0, 0)),
        ],
        out_specs=[
            pl.BlockSpec((512, D), lambda i: (i, 0)),
            pl.BlockSpec((512, 2), lambda i: (i, 0)),
        ],
        out_shape=[
            jax.ShapeDtypeStruct((NP, D), jnp.float32),
            jax.ShapeDtypeStruct((NP, 2), jnp.float32),
        ],
    )(nd, den, b, wp)


def _mm_last(nd, den, b, wo, bo):
    return pl.pallas_call(
        _mm_last_body,
        grid=(NP // 512,),
        in_specs=[
            pl.BlockSpec((2, 512, D), lambda i: (0, i, 0)),
            pl.BlockSpec((NW, 512), lambda i: (0, i)),
            pl.BlockSpec((1, D), lambda i: (0, 0)),
            pl.BlockSpec((D, 16), lambda i: (0, 0)),
            pl.BlockSpec((1, 16), lambda i: (0, 0)),
        ],
        out_specs=pl.BlockSpec((512, 16), lambda i: (i, 0)),
        out_shape=jax.ShapeDtypeStruct((NP, 16), jnp.float32),
    )(nd, den, b, wo, bo)


@functools.partial(
    pl.kernel,
    out_type=[
        jax.ShapeDtypeStruct((2, NP, D), jnp.float32),
        jax.ShapeDtypeStruct((NW, NP), jnp.float32),
    ],
    mesh=_mesh,
    compiler_params=pltpu.CompilerParams(
        needs_layout_passes=False, use_tc_tiling_on_sc=False),
    scratch_types=[
        pltpu.VMEM((NA,), jnp.float32),
        pltpu.VMEM((NP,), jnp.float32),
        pltpu.VMEM((G, CH), jnp.int32),
        pltpu.VMEM((G, CH), jnp.int32),
        pltpu.VMEM((G, CH), jnp.int32),
        pltpu.VMEM((G, CH), jnp.int32),
        pltpu.VMEM((CH, D), jnp.float32),
        pltpu.VMEM((CH, D), jnp.float32),
        pltpu.VMEM((CH,), jnp.float32),
        pltpu.VMEM_SHARED((NP, D), jnp.float32),
        pltpu.SemaphoreType.DMA,
        pltpu.SemaphoreType.DMA,
        pltpu.SemaphoreType.DMA,
        pltpu.SemaphoreType.DMA,
        pltpu.SemaphoreType.DMA,
        pltpu.SemaphoreType.DMA,
    ],
)
def _sc_edge(al_hbm, src_hbm, dst_hbm, h_hbm, z_hbm, num_out, den_out,
             al_v, den_t, srcA, dstA, srcB, dstB, rows_e, rows_o, exb,
             num_sp, gsem_e, gsem_o, ssem_e, ssem_o, isemA, isemB):
    c = lax.axis_index("c")
    s = lax.axis_index("s")
    wid = s * 2 + c
    base = wid * WROWS
    zf16 = jnp.zeros((16,), jnp.float32)

    pltpu.sync_copy(al_hbm, al_v)
    pltpu.sync_copy(z_hbm.at[pl.ds(s * RPT, RPT)],
                    num_sp.at[pl.ds(s * RPT, RPT)])

    def zden_body(i, carry):
        den_t[pl.ds(i * 16, 16)] = zf16
        return carry

    lax.fori_loop(0, NP // 16, zden_body, 0)
    plsc.subcore_barrier()

    def _sub_iter(r, i_src, i_dst, rows_x, gsem_x, ssem_x,
                  rows_y, gsem_y, ssem_y, n_src, n_sem, wait_y, wait_next,
                  use_next, next_ok):
        pltpu.make_async_copy(h_hbm.at[i_src.at[0]], rows_x, gsem_x).wait()

        @pl.when(wait_y)
        def _():
            pltpu.make_async_copy(
                rows_y, num_sp.at[i_dst.at[0]], ssem_y).wait()

        @pl.when(wait_next)
        def _():
            pltpu.make_async_copy(
                src_hbm.at[pl.ds(0, G)], n_src[0], n_sem).wait()
            pltpu.make_async_copy(
                src_hbm.at[pl.ds(0, G)], n_src[1], n_sem).wait()

        @pl.when(next_ok & jnp.logical_not(use_next))
        def _():
            pltpu.async_copy(h_hbm.at[i_src.at[jnp.minimum(r + 1, G - 1)]],
                             rows_y, gsem_y)

        @pl.when(next_ok & use_next)
        def _():
            pltpu.async_copy(h_hbm.at[n_src[0].at[0]], rows_y, gsem_y)

        def ex_body(g, carry2):
            srcv = i_src[r, pl.ds(g * 16, 16)]
            dstv = i_dst[r, pl.ds(g * 16, 16)]
            dstv2 = dstv * 2
            as_s = plsc.load_gather(al_v, [srcv * 2])
            as_d = plsc.load_gather(al_v, [dstv2])
            ad_d = plsc.load_gather(al_v, [dstv2 + 1])
            e = as_s + ad_d
            e = jnp.maximum(e, 0.2 * e)
            m = as_d + ad_d
            m = jnp.maximum(m, 0.2 * m)
            exv = jnp.exp(e - m)
            exb[pl.ds(g * 16, 16)] = exv
            plsc.addupdate_scatter(den_t, [dstv], exv)
            return carry2

        lax.fori_loop(0, NG, ex_body, 0)

        def sc_body(g, carry2):
            exv16 = exb[pl.ds(g * 16, 16)]
            for l in range(16):
                i = g * 16 + l
                exq = jnp.full((16,), exv16[l], jnp.float32)
                for f in range(8):
                    rows_x[i, pl.ds(f * 16, 16)] = (
                        rows_x[i, pl.ds(f * 16, 16)] * exq)
            return carry2

        lax.fori_loop(0, NG, sc_body, 0)
        pltpu.async_copy(rows_x, num_sp.at[i_dst.at[r]], ssem_x, add=True)

    def _block(i_src, i_dst, nxt, n_sem, wait0, refill_pending, next_ok,
               refill_row, refill_guard):
        true_ = jnp.bool_(True)

        def duo(rp, carry2):
            r = rp * 2
            last = rp >= G // 2 - 1
            _sub_iter(r, i_src, i_dst, rows_e, gsem_e, ssem_e,
                      rows_o, gsem_o, ssem_o, nxt, n_sem,
                      jnp.logical_or(rp > 0, wait0),
                      jnp.logical_and(last, refill_pending),
                      jnp.bool_(False), true_)

            @pl.when(jnp.logical_and(rp == 1, refill_guard))
            def _():
                pltpu.async_copy(
                    src_hbm.at[pl.ds(refill_row, G)], nxt[0], n_sem)
                pltpu.async_copy(
                    dst_hbm.at[pl.ds(refill_row, G)], nxt[1], n_sem)

            _sub_iter(r + 1, i_src, i_dst, rows_o, gsem_o, ssem_o,
                      rows_e, gsem_e, ssem_e, nxt, n_sem,
                      true_, jnp.bool_(False), last,
                      jnp.logical_or(jnp.logical_not(last), next_ok))
            return carry2

        lax.fori_loop(0, G // 2, duo, 0)

    QMAX = NBLK // 2 - 1

    def pair_body(q, carry):
        blkA = base + q * 2 * G
        _block(srcA, dstA, (srcB, dstB), isemB, q > 0,
               jnp.bool_(True), jnp.bool_(True), blkA + G, q > 0)
        _block(srcB, dstB, (srcA, dstA), isemA, jnp.bool_(True),
               q < QMAX, q < QMAX, blkA + 2 * G, q < QMAX)
        return carry

    pltpu.sync_copy(src_hbm.at[pl.ds(base, G)], srcA)
    pltpu.sync_copy(dst_hbm.at[pl.ds(base, G)], dstA)
    pltpu.async_copy(src_hbm.at[pl.ds(base + G, G)], srcB, isemB)
    pltpu.async_copy(dst_hbm.at[pl.ds(base + G, G)], dstB, isemB)
    pltpu.async_copy(h_hbm.at[srcA.at[0]], rows_e, gsem_e)

    lax.fori_loop(0, NBLK // 2, pair_body, 0)
    pltpu.make_async_copy(rows_o, num_sp.at[dstB.at[0]], ssem_o).wait()
    plsc.subcore_barrier()
    pltpu.sync_copy(num_sp.at[pl.ds(s * RPT, RPT)],
                    num_out.at[c, pl.ds(s * RPT, RPT)])
    pltpu.sync_copy(den_t.at[pl.ds(0, NP)], den_out.at[wid])


def kernel(x, edge_index, W1, a_src1, a_dst1, b1, W2, a_src2, a_dst2, b2,
           Wf, bf, Ws, bs):
    f32 = jnp.float32
    xp = jnp.zeros((NP, D), f32).at[:N].set(x)

    def augment(W, a_src, a_dst):
        return jnp.concatenate(
            [W, (W @ a_src)[:, None], (W @ a_dst)[:, None],
             jnp.zeros((D, DA - D - 2), f32)], axis=1)

    w1p = augment(W1, a_src1, a_dst1)
    w2p = augment(W2, a_src2, a_dst2)
    wo = jnp.concatenate([Wf, Ws, jnp.zeros((D, 6), f32)], axis=1)
    bo = jnp.concatenate([bf, bs, jnp.zeros((6,), f32)])[None, :]

    sl = jnp.arange(N, dtype=jnp.int32)
    pad = jnp.full((EP - ET,), N, jnp.int32)
    src2d = jnp.concatenate([edge_index[0], sl, pad]).reshape(EROWS, CH)
    dst2d = jnp.concatenate([edge_index[1], sl, pad]).reshape(EROWS, CH)
    znd = jnp.zeros((NP, D), f32)

    h1, al1 = _mm_first(xp, w1p)
    nd1, den1 = _sc_edge(al1.reshape(2 * NP)[:NA], src2d, dst2d, h1, znd)
    h2, al2 = _mm_mid(nd1, den1, b1[None, :], w2p)
    nd2, den2 = _sc_edge(al2.reshape(2 * NP)[:NA], src2d, dst2d, h2, znd)
    out = _mm_last(nd2, den2, b2[None, :], wo, bo)
    return (out[:N, :3], out[:N, 3:10])


# speedup vs baseline: 1.0002x; 1.0001x over previous
import functools

import jax
import jax.numpy as jnp
from jax import lax
from jax.experimental import pallas as pl
from jax.experimental.pallas import tpu as pltpu
from jax.experimental.pallas import tpu_sc as plsc

N = 10000
NP = 10240
D = 128
DA = 136
E = 320000
ET = E + N
CH = 64
G = 4
NBLK = 44
WROWS = G * NBLK
NW = 32
EROWS = NW * WROWS
EP = EROWS * CH
NG = CH // 16
NA = 2 * 10016
RPT = NP // 16

_mesh = plsc.VectorSubcoreMesh(core_axis_name="c", subcore_axis_name="s")


def _mm_first_body(x_ref, w_ref, h_ref, al_ref):
    h = jnp.dot(x_ref[...], w_ref[...], preferred_element_type=jnp.float32)
    h_ref[...] = h[:, :D]
    al_ref[...] = h[:, D:D + 2]


def _finish_layer(nd_ref, den_ref, b_ref):
    t = nd_ref[0] + nd_ref[1]
    ones = jnp.ones((NW, 1), jnp.float32)
    dsum = lax.dot_general(den_ref[...], ones, (((0,), (0,)), ((), ())),
                           preferred_element_type=jnp.float32)
    den = jnp.maximum(dsum, 1e-30)
    return jnp.maximum(t / den + b_ref[...], 0.0)


def _mm_mid_body(nd_ref, den_ref, b_ref, w_ref, h_ref, al_ref):
    x2 = _finish_layer(nd_ref, den_ref, b_ref)
    h = jnp.dot(x2, w_ref[...], preferred_element_type=jnp.float32)
    h_ref[...] = h[:, :D]
    al_ref[...] = h[:, D:D + 2]


def _mm_last_body(nd_ref, den_ref, b_ref, w_ref, bo_ref, o_ref):
    x3 = _finish_layer(nd_ref, den_ref, b_ref)
    o_ref[...] = (
        jnp.dot(x3, w_ref[...], preferred_element_type=jnp.float32)
        + bo_ref[...]
    )


def _mm_first(xp, wp):
    return pl.pallas_call(
        _mm_first_body,
        grid=(NP // 512,),
        in_specs=[
            pl.BlockSpec((512, D), lambda i: (i, 0)),
            pl.BlockSpec((D, DA), lambda i: (0, 0)),
        ],
        out_specs=[
            pl.BlockSpec((512, D), lambda i: (i, 0)),
            pl.BlockSpec((512, 2), lambda i: (i, 0)),
        ],
        out_shape=[
            jax.ShapeDtypeStruct((NP, D), jnp.float32),
            jax.ShapeDtypeStruct((NP, 2), jnp.float32),
        ],
    )(xp, wp)


def _mm_mid(nd, den, b, wp):
    return pl.pallas_call(
        _mm_mid_body,
        grid=(NP // 512,),
        in_specs=[
            pl.BlockSpec((2, 512, D), lambda i: (0, i, 0)),
            pl.BlockSpec((NW, 512), lambda i: (0, i)),
            pl.BlockSpec((1, D), lambda i: (0, 0)),
            pl.BlockSpec((D, DA), lambda i: (0, 0)),
        ],
        out_specs=[
            pl.BlockSpec((512, D), lambda i: (i, 0)),
            pl.BlockSpec((512, 2), lambda i: (i, 0)),
        ],
        out_shape=[
            jax.ShapeDtypeStruct((NP, D), jnp.float32),
            jax.ShapeDtypeStruct((NP, 2), jnp.float32),
        ],
    )(nd, den, b, wp)


def _mm_last(nd, den, b, wo, bo):
    return pl.pallas_call(
        _mm_last_body,
        grid=(NP // 512,),
        in_specs=[
            pl.BlockSpec((2, 512, D), lambda i: (0, i, 0)),
            pl.BlockSpec((NW, 512), lambda i: (0, i)),
            pl.BlockSpec((1, D), lambda i: (0, 0)),
            pl.BlockSpec((D, 16), lambda i: (0, 0)),
            pl.BlockSpec((1, 16), lambda i: (0, 0)),
        ],
        out_specs=pl.BlockSpec((512, 16), lambda i: (i, 0)),
        out_shape=jax.ShapeDtypeStruct((NP, 16), jnp.float32),
    )(nd, den, b, wo, bo)


@functools.partial(
    pl.kernel,
    out_type=[
        jax.ShapeDtypeStruct((2, NP, D), jnp.float32),
        jax.ShapeDtypeStruct((NW, NP), jnp.float32),
    ],
    mesh=_mesh,
    compiler_params=pltpu.CompilerParams(
        needs_layout_passes=False, use_tc_tiling_on_sc=False),
    scratch_types=[
        pltpu.VMEM((NA,), jnp.float32),
        pltpu.VMEM((NP,), jnp.float32),
        pltpu.VMEM((G, CH), jnp.int32),
        pltpu.VMEM((G, CH), jnp.int32),
        pltpu.VMEM((G, CH), jnp.int32),
        pltpu.VMEM((G, CH), jnp.int32),
        pltpu.VMEM((CH, D), jnp.float32),
        pltpu.VMEM((CH, D), jnp.float32),
        pltpu.VMEM((CH,), jnp.float32),
        pltpu.VMEM_SHARED((NP, D), jnp.float32),
        pltpu.SemaphoreType.DMA,
        pltpu.SemaphoreType.DMA,
        pltpu.SemaphoreType.DMA,
        pltpu.SemaphoreType.DMA,
        pltpu.SemaphoreType.DMA,
        pltpu.SemaphoreType.DMA,
    ],
)
def _sc_edge(al_hbm, src_hbm, dst_hbm, h_hbm, z_hbm, num_out, den_out,
             al_v, den_t, srcA, dstA, srcB, dstB, rows_e, rows_o, exb,
             num_sp, gsem_e, gsem_o, ssem_e, ssem_o, isemA, isemB):
    c = lax.axis_index("c")
    s = lax.axis_index("s")
    wid = s * 2 + c
    base = wid * WROWS
    zf16 = jnp.zeros((16,), jnp.float32)

    pltpu.sync_copy(al_hbm, al_v)
    pltpu.sync_copy(z_hbm.at[pl.ds(s * RPT, RPT)],
                    num_sp.at[pl.ds(s * RPT, RPT)])

    def zden_body(i, carry):
        den_t[pl.ds(i * 16, 16)] = zf16
        return carry

    lax.fori_loop(0, NP // 16, zden_body, 0)
    plsc.subcore_barrier()

    def _sub_iter(r, i_src, i_dst, rows_x, gsem_x, ssem_x,
                  rows_y, gsem_y, ssem_y, n_src, wait_y, next_ok):
        def ex_body(g, carry2):
            srcv = i_src[r, pl.ds(g * 16, 16)]
            dstv = i_dst[r, pl.ds(g * 16, 16)]
            dstv2 = dstv * 2
            as_s = plsc.load_gather(al_v, [srcv * 2])
            as_d = plsc.load_gather(al_v, [dstv2])
            ad_d = plsc.load_gather(al_v, [dstv2 + 1])
            e = as_s + ad_d
            e = jnp.maximum(e, 0.2 * e)
            m = as_d + ad_d
            m = jnp.maximum(m, 0.2 * m)
            exv = jnp.exp(e - m)
            exb[pl.ds(g * 16, 16)] = exv
            plsc.addupdate_scatter(den_t, [dstv], exv)
            return carry2

        lax.fori_loop(0, NG, ex_body, 0)
        pltpu.make_async_copy(h_hbm.at[i_src.at[0]], rows_x, gsem_x).wait()

        if wait_y is not None:
            @pl.when(wait_y)
            def _():
                pltpu.make_async_copy(
                    rows_y, num_sp.at[i_dst.at[0]], ssem_y).wait()

        nidx = i_src.at[r + 1] if r + 1 < G else n_src.at[0]

        @pl.when(next_ok)
        def _():
            pltpu.async_copy(h_hbm.at[nidx], rows_y, gsem_y)

        def sc_body(g, carry2):
            exv16 = exb[pl.ds(g * 16, 16)]
            for l in range(16):
                i = g * 16 + l
                exq = jnp.full((16,), exv16[l], jnp.float32)
                for f in range(8):
                    rows_x[i, pl.ds(f * 16, 16)] = (
                        rows_x[i, pl.ds(f * 16, 16)] * exq)
            return carry2

        lax.fori_loop(0, NG, sc_body, 0)
        pltpu.async_copy(rows_x, num_sp.at[i_dst.at[r]], ssem_x, add=True)

    TRUE = None

    def _block(i_src, i_dst, nxt, n_sem, wait0, refill_pending, next_ok,
               refill_row, refill_guard):
        bufs = ((rows_e, gsem_e, ssem_e), (rows_o, gsem_o, ssem_o))
        for r in range(G):
            bx = bufs[r % 2]
            by = bufs[1 - r % 2]
            _sub_iter(r, i_src, i_dst, bx[0], bx[1], bx[2],
                      by[0], by[1], by[2], nxt[0],
                      wait0 if r == 0 else jnp.bool_(True),
                      next_ok if r == G - 1 else jnp.bool_(True))
            if r == 1:
                @pl.when(refill_guard)
                def _():
                    pltpu.async_copy(
                        src_hbm.at[pl.ds(refill_row, G)], nxt[0], n_sem)
                    pltpu.async_copy(
                        dst_hbm.at[pl.ds(refill_row, G)], nxt[1], n_sem)
            if r == 2:
                @pl.when(refill_pending)
                def _():
                    pltpu.make_async_copy(
                        src_hbm.at[pl.ds(0, G)], nxt[0], n_sem).wait()
                    pltpu.make_async_copy(
                        src_hbm.at[pl.ds(0, G)], nxt[1], n_sem).wait()

    QMAX = NBLK // 2 - 1

    def pair_body(q, carry):
        blkA = base + q * 2 * G
        _block(srcA, dstA, (srcB, dstB), isemB, q > 0,
               jnp.bool_(True), jnp.bool_(True), blkA + G, q > 0)
        _block(srcB, dstB, (srcA, dstA), isemA, jnp.bool_(True),
               q < QMAX, q < QMAX, blkA + 2 * G, q < QMAX)
        return carry

    pltpu.sync_copy(src_hbm.at[pl.ds(base, G)], srcA)
    pltpu.sync_copy(dst_hbm.at[pl.ds(base, G)], dstA)
    pltpu.async_copy(src_hbm.at[pl.ds(base + G, G)], srcB, isemB)
    pltpu.async_copy(dst_hbm.at[pl.ds(base + G, G)], dstB, isemB)
    pltpu.async_copy(h_hbm.at[srcA.at[0]], rows_e, gsem_e)

    lax.fori_loop(0, NBLK // 2, pair_body, 0)
    pltpu.make_async_copy(rows_o, num_sp.at[dstB.at[0]], ssem_o).wait()
    plsc.subcore_barrier()
    pltpu.sync_copy(num_sp.at[pl.ds(s * RPT, RPT)],
                    num_out.at[c, pl.ds(s * RPT, RPT)])
    pltpu.sync_copy(den_t.at[pl.ds(0, NP)], den_out.at[wid])


def kernel(x, edge_index, W1, a_src1, a_dst1, b1, W2, a_src2, a_dst2, b2,
           Wf, bf, Ws, bs):
    f32 = jnp.float32
    xp = jnp.zeros((NP, D), f32).at[:N].set(x)

    def augment(W, a_src, a_dst):
        return jnp.concatenate(
            [W, (W @ a_src)[:, None], (W @ a_dst)[:, None],
             jnp.zeros((D, DA - D - 2), f32)], axis=1)

    w1p = augment(W1, a_src1, a_dst1)
    w2p = augment(W2, a_src2, a_dst2)
    wo = jnp.concatenate([Wf, Ws, jnp.zeros((D, 6), f32)], axis=1)
    bo = jnp.concatenate([bf, bs, jnp.zeros((6,), f32)])[None, :]

    sl = jnp.arange(N, dtype=jnp.int32)
    pad = jnp.full((EP - ET,), N, jnp.int32)
    src2d = jnp.concatenate([edge_index[0], sl, pad]).reshape(EROWS, CH)
    dst2d = jnp.concatenate([edge_index[1], sl, pad]).reshape(EROWS, CH)
    znd = jnp.zeros((NP, D), f32)

    h1, al1 = _mm_first(xp, w1p)
    nd1, den1 = _sc_edge(al1.reshape(2 * NP)[:NA], src2d, dst2d, h1, znd)
    h2, al2 = _mm_mid(nd1, den1, b1[None, :], w2p)
    nd2, den2 = _sc_edge(al2.reshape(2 * NP)[:NA], src2d, dst2d, h2, znd)
    out = _mm_last(nd2, den2, b2[None, :], wo, bo)
    return (out[:N, :3], out[:N, 3:10])


# speedup vs baseline: 2.5193x; 2.5188x over previous
import functools

import jax
import jax.numpy as jnp
from jax import lax
from jax.experimental import pallas as pl
from jax.experimental.pallas import tpu as pltpu
from jax.experimental.pallas import tpu_sc as plsc

N = 10000
NP = 10240
D = 128
DA = 136
E = 320000
ET = E + N
CH = 64
NW = 32
EROWS = 5248
WROWS = EROWS // NW
EP = EROWS * CH
NG = CH // 16
NA = 2 * 10016
RPT = NP // 16

_mesh = plsc.VectorSubcoreMesh(core_axis_name="c", subcore_axis_name="s")


def _mm_first_body(x_ref, w_ref, h_ref, al_ref):
    h = jnp.dot(x_ref[...], w_ref[...], preferred_element_type=jnp.float32)
    h_ref[...] = h[:, :D]
    al_ref[...] = h[:, D:D + 2]


def _finish_layer(nd_ref, den_ref, b_ref):
    t = nd_ref[0] + nd_ref[1]
    ones = jnp.ones((NW, 1), jnp.float32)
    dsum = lax.dot_general(den_ref[...], ones, (((0,), (0,)), ((), ())),
                           preferred_element_type=jnp.float32)
    den = jnp.maximum(dsum, 1e-30)
    return jnp.maximum(t / den + b_ref[...], 0.0)


def _mm_mid_body(nd_ref, den_ref, b_ref, w_ref, h_ref, al_ref):
    x2 = _finish_layer(nd_ref, den_ref, b_ref)
    h = jnp.dot(x2, w_ref[...], preferred_element_type=jnp.float32)
    h_ref[...] = h[:, :D]
    al_ref[...] = h[:, D:D + 2]


def _mm_last_body(nd_ref, den_ref, b_ref, w_ref, bo_ref, o_ref):
    x3 = _finish_layer(nd_ref, den_ref, b_ref)
    o_ref[...] = (
        jnp.dot(x3, w_ref[...], preferred_element_type=jnp.float32)
        + bo_ref[...]
    )


def _mm_first(xp, wp):
    return pl.pallas_call(
        _mm_first_body,
        grid=(NP // 512,),
        in_specs=[
            pl.BlockSpec((512, D), lambda i: (i, 0)),
            pl.BlockSpec((D, DA), lambda i: (0, 0)),
        ],
        out_specs=[
            pl.BlockSpec((512, D), lambda i: (i, 0)),
            pl.BlockSpec((512, 2), lambda i: (i, 0)),
        ],
        out_shape=[
            jax.ShapeDtypeStruct((NP, D), jnp.float32),
            jax.ShapeDtypeStruct((NP, 2), jnp.float32),
        ],
    )(xp, wp)


def _mm_mid(nd, den, b, wp):
    return pl.pallas_call(
        _mm_mid_body,
        grid=(NP // 512,),
        in_specs=[
            pl.BlockSpec((2, 512, D), lambda i: (0, i, 0)),
            pl.BlockSpec((NW, 512), lambda i: (0, i)),
            pl.BlockSpec((1, D), lambda i: (0, 0)),
            pl.BlockSpec((D, DA), lambda i: (0, 0)),
        ],
        out_specs=[
            pl.BlockSpec((512, D), lambda i: (i, 0)),
            pl.BlockSpec((512, 2), lambda i: (i, 0)),
        ],
        out_shape=[
            jax.ShapeDtypeStruct((NP, D), jnp.float32),
            jax.ShapeDtypeStruct((NP, 2), jnp.float32),
        ],
    )(nd, den, b, wp)


def _mm_last(nd, den, b, wo, bo):
    return pl.pallas_call(
        _mm_last_body,
        grid=(NP // 512,),
        in_specs=[
            pl.BlockSpec((2, 512, D), lambda i: (0, i, 0)),
            pl.BlockSpec((NW, 512), lambda i: (0, i)),
            pl.BlockSpec((1, D), lambda i: (0, 0)),
            pl.BlockSpec((D, 16), lambda i: (0, 0)),
            pl.BlockSpec((1, 16), lambda i: (0, 0)),
        ],
        out_specs=pl.BlockSpec((512, 16), lambda i: (i, 0)),
        out_shape=jax.ShapeDtypeStruct((NP, 16), jnp.float32),
    )(nd, den, b, wo, bo)


@functools.partial(
    pl.kernel,
    out_type=[
        jax.ShapeDtypeStruct((2, NP, D), jnp.float32),
        jax.ShapeDtypeStruct((NW, NP), jnp.float32),
    ],
    mesh=_mesh,
    compiler_params=pltpu.CompilerParams(
        needs_layout_passes=False, use_tc_tiling_on_sc=False),
    scratch_types=[
        pltpu.VMEM((NA,), jnp.float32),
        pltpu.VMEM((NP,), jnp.float32),
        pltpu.VMEM((1, CH), jnp.int32),
        pltpu.VMEM((1, CH), jnp.int32),
        pltpu.VMEM((1, CH), jnp.int32),
        pltpu.VMEM((1, CH), jnp.int32),
        pltpu.VMEM((CH, D), jnp.float32),
        pltpu.VMEM((CH, D), jnp.float32),
        pltpu.VMEM((CH,), jnp.float32),
        pltpu.VMEM_SHARED((NP, D), jnp.float32),
        pltpu.SemaphoreType.DMA,
        pltpu.SemaphoreType.DMA,
        pltpu.SemaphoreType.DMA,
        pltpu.SemaphoreType.DMA,
    ],
)
def _sc_edge(al_hbm, src_hbm, dst_hbm, h_hbm, z_hbm, num_out, den_out,
             al_v, den_t, srcb_e, dstb_e, srcb_o, dstb_o, rows_e, rows_o,
             exb, num_sp, gsem_e, gsem_o, ssem_e, ssem_o):
    c = lax.axis_index("c")
    s = lax.axis_index("s")
    wid = s * 2 + c
    base = wid * WROWS
    zf16 = jnp.zeros((16,), jnp.float32)

    pltpu.sync_copy(al_hbm, al_v)
    pltpu.sync_copy(z_hbm.at[pl.ds(s * RPT, RPT)],
                    num_sp.at[pl.ds(s * RPT, RPT)])

    def zden_body(i, carry):
        den_t[pl.ds(i * 16, 16)] = zf16
        return carry

    lax.fori_loop(0, NP // 16, zden_body, 0)
    plsc.subcore_barrier()

    def _sub_iter(cur, srcb_x, dstb_x, rows_x, gsem_x, ssem_x,
                  srcb_y, dstb_y, rows_y, gsem_y, ssem_y, wait_y, do_pref):
        def ex_body(g, carry2):
            srcv = srcb_x[0, pl.ds(g * 16, 16)]
            dstv = dstb_x[0, pl.ds(g * 16, 16)]
            dstv2 = dstv * 2
            as_s = plsc.load_gather(al_v, [srcv * 2])
            as_d = plsc.load_gather(al_v, [dstv2])
            ad_d = plsc.load_gather(al_v, [dstv2 + 1])
            e = as_s + ad_d
            e = jnp.maximum(e, 0.2 * e)
            m = as_d + ad_d
            m = jnp.maximum(m, 0.2 * m)
            exv = jnp.exp(e - m)
            exb[pl.ds(g * 16, 16)] = exv
            plsc.addupdate_scatter(den_t, [dstv], exv)
            return carry2

        lax.fori_loop(0, NG, ex_body, 0)

        @pl.when(wait_y)
        def _():
            pltpu.make_async_copy(
                rows_y, num_sp.at[dstb_y.at[0]], ssem_y).wait()

        @pl.when(do_pref)
        def _():
            pltpu.sync_copy(src_hbm.at[pl.ds(cur + 1, 1)], srcb_y)
            pltpu.sync_copy(dst_hbm.at[pl.ds(cur + 1, 1)], dstb_y)

        pltpu.make_async_copy(h_hbm.at[srcb_x.at[0]], rows_x, gsem_x).wait()

        @pl.when(do_pref)
        def _():
            pltpu.async_copy(h_hbm.at[srcb_y.at[0]], rows_y, gsem_y)

        def sc_body(g, carry2):
            exv16 = exb[pl.ds(g * 16, 16)]
            for l in range(16):
                i = g * 16 + l
                exq = jnp.full((16,), exv16[l], jnp.float32)
                for f in range(8):
                    rows_x[i, pl.ds(f * 16, 16)] = (
                        rows_x[i, pl.ds(f * 16, 16)] * exq)
            return carry2

        lax.fori_loop(0, NG, sc_body, 0)
        pltpu.async_copy(rows_x, num_sp.at[dstb_x.at[0]], ssem_x, add=True)

    def pair_body(p, carry):
        ce = base + 2 * p
        _sub_iter(ce, srcb_e, dstb_e, rows_e, gsem_e, ssem_e,
                  srcb_o, dstb_o, rows_o, gsem_o, ssem_o,
                  p > 0, p >= 0)
        _sub_iter(ce + 1, srcb_o, dstb_o, rows_o, gsem_o, ssem_o,
                  srcb_e, dstb_e, rows_e, gsem_e, ssem_e,
                  p >= 0, p < WROWS // 2 - 1)
        return carry

    pltpu.sync_copy(src_hbm.at[pl.ds(base, 1)], srcb_e)
    pltpu.sync_copy(dst_hbm.at[pl.ds(base, 1)], dstb_e)
    pltpu.async_copy(h_hbm.at[srcb_e.at[0]], rows_e, gsem_e)

    lax.fori_loop(0, WROWS // 2, pair_body, 0)
    pltpu.make_async_copy(rows_o, num_sp.at[dstb_o.at[0]], ssem_o).wait()
    plsc.subcore_barrier()
    pltpu.sync_copy(num_sp.at[pl.ds(s * RPT, RPT)],
                    num_out.at[c, pl.ds(s * RPT, RPT)])
    pltpu.sync_copy(den_t.at[pl.ds(0, NP)], den_out.at[wid])


def kernel(x, edge_index, W1, a_src1, a_dst1, b1, W2, a_src2, a_dst2, b2,
           Wf, bf, Ws, bs):
    f32 = jnp.float32
    xp = jnp.zeros((NP, D), f32).at[:N].set(x)

    def augment(W, a_src, a_dst):
        return jnp.concatenate(
            [W, (W @ a_src)[:, None], (W @ a_dst)[:, None],
             jnp.zeros((D, DA - D - 2), f32)], axis=1)

    w1p = augment(W1, a_src1, a_dst1)
    w2p = augment(W2, a_src2, a_dst2)
    wo = jnp.concatenate([Wf, Ws, jnp.zeros((D, 6), f32)], axis=1)
    bo = jnp.concatenate([bf, bs, jnp.zeros((6,), f32)])[None, :]

    sl = jnp.arange(N, dtype=jnp.int32)
    pad = jnp.full((EP - ET,), N, jnp.int32)
    src2d = jnp.concatenate([edge_index[0], sl, pad]).reshape(EROWS, CH)
    dst2d = jnp.concatenate([edge_index[1], sl, pad]).reshape(EROWS, CH)
    znd = jnp.zeros((NP, D), f32)

    h1, al1 = _mm_first(xp, w1p)
    nd1, den1 = _sc_edge(al1.reshape(2 * NP)[:NA], src2d, dst2d, h1, znd)
    h2, al2 = _mm_mid(nd1, den1, b1[None, :], w2p)
    nd2, den2 = _sc_edge(al2.reshape(2 * NP)[:NA], src2d, dst2d, h2, znd)
    out = _mm_last(nd2, den2, b2[None, :], wo, bo)
    return (out[:N, :3], out[:N, 3:10])
